# baseline (speedup 1.0000x reference)
import jax
import jax.numpy as jnp
from jax import lax
from jax.experimental import pallas as pl
from jax.experimental.pallas import tpu as pltpu
from jax.experimental.pallas import tpu_sc as plsc

_S = 30.0
_BLKC = 8192

_SC_WORKERS = 32
_SC_LANES = 16
_PAD_C = 112


def _margin_gather_sc(tgt2, base_m_list, class_margin_weights, batch):
    per_w = batch // _SC_WORKERS
    c = base_m_list.shape[0]
    mesh = plsc.VectorSubcoreMesh(core_axis_name="c", subcore_axis_name="s")

    @pl.kernel(
        out_type=jax.ShapeDtypeStruct((1, batch), jnp.float32),
        mesh=mesh,
        compiler_params=pltpu.CompilerParams(needs_layout_passes=False),
        scratch_types=[
            pltpu.VMEM((per_w,), jnp.int32),
            pltpu.VMEM((_PAD_C,), jnp.float32),
            pltpu.VMEM((_PAD_C,), jnp.float32),
            pltpu.VMEM((_PAD_C,), jnp.float32),
            pltpu.VMEM((per_w,), jnp.float32),
        ],
    )
    def sc_kernel(t_hbm, bm_hbm, w_hbm, out_hbm, t_v, bm_v, w_v, m_v, out_v):
        wid = lax.axis_index("s") * 2 + lax.axis_index("c")
        base = wid * per_w
        pltpu.sync_copy(t_hbm.at[0, pl.ds(base, per_w)], t_v)
        pltpu.sync_copy(bm_hbm, bm_v.at[pl.ds(0, c)])
        pltpu.sync_copy(w_hbm, w_v.at[pl.ds(0, c)])

        @pl.loop(0, _PAD_C, step=_SC_LANES)
        def _(j):
            wv = w_v[pl.ds(j, _SC_LANES)]
            sig = 1.0 / (1.0 + jnp.exp(-wv))
            m_v[pl.ds(j, _SC_LANES)] = bm_v[pl.ds(j, _SC_LANES)] * sig

        @pl.loop(0, per_w, step=_SC_LANES)
        def _(j):
            idx = t_v[pl.ds(j, _SC_LANES)]
            out_v[pl.ds(j, _SC_LANES)] = plsc.load_gather(m_v, [idx])

        pltpu.sync_copy(out_v, out_hbm.at[0, pl.ds(base, per_w)])

    return sc_kernel(tgt2, base_m_list, class_margin_weights)


def _stats_body(logit_ref, tgt_ref, stats_ref):
    x = jnp.transpose(logit_ref[...])
    t = tgt_ref[...]

    cls = jax.lax.broadcasted_iota(jnp.int32, x.shape, 0)
    onehot = cls == t
    picked = jnp.sum(jnp.where(onehot, x, 0.0), axis=0, keepdims=True)
    mx = jnp.max(x, axis=0, keepdims=True)
    z = jnp.sum(jnp.exp(x - mx), axis=0, keepdims=True)
    stats_ref[...] = jnp.concatenate([z, picked - mx], axis=0)


def _combine_body(stats_ref, mcol_ref, out_ref):
    st = stats_ref[...]
    z = st[0:1, :]
    u = st[1:2, :]
    sm = _S * mcol_ref[...]
    zp = z - jnp.exp(u) + jnp.exp(u - sm)
    loss = jnp.log(zp) - u + sm
    out_ref[...] = jnp.sum(loss, axis=1, keepdims=True) / loss.shape[1]


def kernel(logit, target, base_m_list, class_margin_weights):
    b, c = logit.shape
    tgt2 = target.reshape(1, b)
    m_g = _margin_gather_sc(tgt2, base_m_list, class_margin_weights, b)

    grid = b // _BLKC
    stats = pl.pallas_call(
        _stats_body,
        grid=(grid,),
        in_specs=[
            pl.BlockSpec((_BLKC, c), lambda i: (i, 0)),
            pl.BlockSpec((1, _BLKC), lambda i: (0, i)),
        ],
        out_specs=pl.BlockSpec((2, _BLKC), lambda i: (0, i)),
        out_shape=jax.ShapeDtypeStruct((2, b), jnp.float32),
    )(logit, tgt2)

    out = pl.pallas_call(
        _combine_body,
        out_shape=jax.ShapeDtypeStruct((1, 1), jnp.float32),
    )(stats, m_g)
    return out[0, 0]


# speedup vs baseline: 1.3348x; 1.3348x over previous
import jax
import jax.numpy as jnp
from jax import lax
from jax.experimental import pallas as pl
from jax.experimental.pallas import tpu as pltpu
from jax.experimental.pallas import tpu_sc as plsc

_S = 30.0
_BLKC = 2048

_SC_WORKERS = 32
_SC_LANES = 16
_PAD_C = 112


def _margin_gather_sc(tgt2, base_m_list, class_margin_weights, batch):
    per_w = batch // _SC_WORKERS
    c = base_m_list.shape[0]
    mesh = plsc.VectorSubcoreMesh(core_axis_name="c", subcore_axis_name="s")

    @pl.kernel(
        out_type=jax.ShapeDtypeStruct((1, batch), jnp.float32),
        mesh=mesh,
        compiler_params=pltpu.CompilerParams(needs_layout_passes=False),
        scratch_types=[
            pltpu.VMEM((per_w,), jnp.int32),
            pltpu.VMEM((_PAD_C,), jnp.float32),
            pltpu.VMEM((_PAD_C,), jnp.float32),
            pltpu.VMEM((_PAD_C,), jnp.float32),
            pltpu.VMEM((per_w,), jnp.float32),
        ],
    )
    def sc_kernel(t_hbm, bm_hbm, w_hbm, out_hbm, t_v, bm_v, w_v, m_v, out_v):
        wid = lax.axis_index("s") * 2 + lax.axis_index("c")
        base = wid * per_w
        pltpu.sync_copy(t_hbm.at[0, pl.ds(base, per_w)], t_v)
        pltpu.sync_copy(bm_hbm, bm_v.at[pl.ds(0, c)])
        pltpu.sync_copy(w_hbm, w_v.at[pl.ds(0, c)])

        @pl.loop(0, _PAD_C, step=_SC_LANES)
        def _(j):
            wv = w_v[pl.ds(j, _SC_LANES)]
            sig = 1.0 / (1.0 + jnp.exp(-wv))
            m_v[pl.ds(j, _SC_LANES)] = bm_v[pl.ds(j, _SC_LANES)] * sig

        @pl.loop(0, per_w, step=_SC_LANES)
        def _(j):
            idx = t_v[pl.ds(j, _SC_LANES)]
            out_v[pl.ds(j, _SC_LANES)] = plsc.load_gather(m_v, [idx])

        pltpu.sync_copy(out_v, out_hbm.at[0, pl.ds(base, per_w)])

    return sc_kernel(tgt2, base_m_list, class_margin_weights)


_NSTREAM = 4


def _stats_body(*refs):
    xs = refs[:_NSTREAM]
    ts = refs[_NSTREAM:2 * _NSTREAM]
    outs = refs[2 * _NSTREAM:]
    for logit_ref, tgt_ref, stats_ref in zip(xs, ts, outs):
        x = logit_ref[...]
        t = tgt_ref[...]
        cls = jax.lax.broadcasted_iota(jnp.int32, x.shape, 0)
        onehot = cls == t
        picked = jnp.sum(jnp.where(onehot, x, 0.0), axis=0, keepdims=True)
        mx = jnp.max(x, axis=0, keepdims=True)
        z = jnp.sum(jnp.exp(x - mx), axis=0, keepdims=True)
        stats_ref[...] = jnp.concatenate([z, picked - mx], axis=0)


def _combine_body(*refs):
    sts = refs[:_NSTREAM]
    mcol_ref = refs[_NSTREAM]
    out_ref = refs[_NSTREAM + 1]
    bpart = sts[0].shape[1]
    total = None
    for k, stats_ref in enumerate(sts):
        st = stats_ref[...]
        z = st[0:1, :]
        u = st[1:2, :]
        sm = _S * mcol_ref[0:1, k * bpart:(k + 1) * bpart]
        zp = z - jnp.exp(u) + jnp.exp(u - sm)
        loss = jnp.log(zp) - u + sm
        s = jnp.sum(loss, axis=1, keepdims=True)
        total = s if total is None else total + s
    out_ref[...] = total / (bpart * _NSTREAM)


def kernel(logit, target, base_m_list, class_margin_weights):
    b, c = logit.shape
    tgt2 = target.reshape(1, b)
    m_g = _margin_gather_sc(tgt2, base_m_list, class_margin_weights, b)

    xt = logit.T
    part = b // _NSTREAM
    grid = part // _BLKC
    off = part // _BLKC

    def x_spec(k):
        return pl.BlockSpec((c, _BLKC), lambda i, k=k: (0, i + k * off))

    def t_spec(k):
        return pl.BlockSpec((1, _BLKC), lambda i, k=k: (0, i + k * off))

    stats = pl.pallas_call(
        _stats_body,
        grid=(grid,),
        in_specs=[x_spec(k) for k in range(_NSTREAM)]
        + [t_spec(k) for k in range(_NSTREAM)],
        out_specs=[pl.BlockSpec((2, _BLKC), lambda i: (0, i))] * _NSTREAM,
        out_shape=[jax.ShapeDtypeStruct((2, part), jnp.float32)] * _NSTREAM,
    )(*([xt] * _NSTREAM + [tgt2] * _NSTREAM))

    out = pl.pallas_call(
        _combine_body,
        out_shape=jax.ShapeDtypeStruct((1, 1), jnp.float32),
    )(*stats, m_g)
    return out[0, 0]
